# baseline (speedup 1.0000x reference)
import functools
import numpy as np

import jax
import jax.numpy as jnp
from jax import lax
from jax.experimental import pallas as pl
from jax.experimental.pallas import tpu as pltpu
from jax.experimental.pallas import tpu_sc as plsc

N_NODES = 50000
N_EDGES = 800000
HID = 64
NUM_GRAPHS = 256
_DEG_HIST = np.array([0, 0, 0, 0, 120, 360, 840, 1600, 2600, 3600, 4400, 4900,
                      5000, 4900, 4400, 3600, 2600, 1600, 840, 360, 120],
                     dtype=np.float64)
AVG_LOG = float((np.log(np.arange(len(_DEG_HIST)) + 1.0) * _DEG_HIST).sum()
                / _DEG_HIST.sum())

_NW = 32
_NV = 1600
_NPAD = _NW * _NV
_RSBUF = 1664
_BLK = 32
_STAGE = 64
_MPAD = N_EDGES + 2 * _BLK + 8


def _seg_agg_body(m_hbm, rs_hbm, mn_hbm, mx_hbm, s_hbm, q_hbm,
                  rows_v, rs_v, mnst, mxst, sst, qst):
    wid = lax.axis_index("s") * 2 + lax.axis_index("c")
    vbase = wid * _NV
    pltpu.sync_copy(rs_hbm.at[pl.ds(vbase, _RSBUF)], rs_v)

    inf = jnp.full((16,), jnp.inf, jnp.float32)
    zero = jnp.zeros((16,), jnp.float32)

    def node_body(v_local, _):
        rsvec = rs_v[pl.ds(v_local, 16)]
        e0 = rsvec[0]
        e1 = rsvec[1]
        eb0 = pl.multiple_of((e0 // 8) * 8, 8)
        nblk = jnp.maximum(e1 - eb0 + (_BLK - 1), 0) // _BLK

        def blk_body(k, acc):
            eb = pl.multiple_of(eb0 + k * _BLK, 8)
            jstart = jnp.maximum(e0 - eb, 0)
            jend = jnp.minimum(e1 - eb, _BLK)
            pltpu.sync_copy(m_hbm.at[pl.ds(eb, _BLK)], rows_v)

            def edge_body(j, acc):
                (mn0, mn1, mn2, mn3, mx0, mx1, mx2, mx3,
                 s0, s1, s2, s3, q0, q1, q2, q3) = acc
                r0 = rows_v[j, pl.ds(0, 16)]
                r1 = rows_v[j, pl.ds(16, 16)]
                r2 = rows_v[j, pl.ds(32, 16)]
                r3 = rows_v[j, pl.ds(48, 16)]
                return (jnp.minimum(mn0, r0), jnp.minimum(mn1, r1),
                        jnp.minimum(mn2, r2), jnp.minimum(mn3, r3),
                        jnp.maximum(mx0, r0), jnp.maximum(mx1, r1),
                        jnp.maximum(mx2, r2), jnp.maximum(mx3, r3),
                        s0 + r0, s1 + r1, s2 + r2, s3 + r3,
                        q0 + r0 * r0, q1 + r1 * r1,
                        q2 + r2 * r2, q3 + r3 * r3)

            return lax.fori_loop(jstart, jend, edge_body, acc)

        init = (inf, inf, inf, inf, -inf, -inf, -inf, -inf,
                zero, zero, zero, zero, zero, zero, zero, zero)
        acc = lax.fori_loop(0, nblk, blk_body, init)

        sr = lax.rem(v_local, _STAGE)
        for k in range(4):
            mnst[sr, pl.ds(16 * k, 16)] = acc[k]
            mxst[sr, pl.ds(16 * k, 16)] = acc[4 + k]
            sst[sr, pl.ds(16 * k, 16)] = acc[8 + k]
            qst[sr, pl.ds(16 * k, 16)] = acc[12 + k]

        @pl.when(sr == _STAGE - 1)
        def _flush():
            row0 = pl.multiple_of(vbase + v_local - (_STAGE - 1), 8)
            pltpu.sync_copy(mnst, mn_hbm.at[pl.ds(row0, _STAGE)])
            pltpu.sync_copy(mxst, mx_hbm.at[pl.ds(row0, _STAGE)])
            pltpu.sync_copy(sst, s_hbm.at[pl.ds(row0, _STAGE)])
            pltpu.sync_copy(qst, q_hbm.at[pl.ds(row0, _STAGE)])

        return _

    lax.fori_loop(0, _NV, node_body, 0)


def _seg_agg(m_sorted, rs_padded):
    out = jax.ShapeDtypeStruct((_NPAD, HID), jnp.float32)
    kern = pl.kernel(
        _seg_agg_body,
        out_type=(out, out, out, out),
        mesh=plsc.VectorSubcoreMesh(core_axis_name="c", subcore_axis_name="s"),
        scratch_types=[
            pltpu.VMEM((_BLK, HID), jnp.float32),
            pltpu.VMEM((_RSBUF,), jnp.int32),
            pltpu.VMEM((_STAGE, HID), jnp.float32),
            pltpu.VMEM((_STAGE, HID), jnp.float32),
            pltpu.VMEM((_STAGE, HID), jnp.float32),
            pltpu.VMEM((_STAGE, HID), jnp.float32),
        ],
    )
    return kern(m_sorted, rs_padded)


def _csr(dst):
    perm = jnp.argsort(dst, stable=True).astype(jnp.int32)
    dst_sorted = dst[perm]
    rs = jnp.searchsorted(dst_sorted,
                          jnp.arange(_NPAD + _RSBUF, dtype=jnp.int32),
                          side='left').astype(jnp.int32)
    return perm, dst_sorted, rs


def _bn(x, g, b):
    mu = x.mean(0)
    v = x.var(0)
    return (x - mu) / jnp.sqrt(v + 1e-5) * g + b


def kernel(x, edge_index, edge_attr, batch, params):
    src, dst = edge_index[0], edge_index[1]
    n = N_NODES
    perm, dst_s, rs = _csr(dst)
    src_s = src[perm]
    h = jax.nn.leaky_relu(x @ params['Wn'] + params['bn'])
    ea = jax.nn.leaky_relu(edge_attr[perm] @ params['We0'] + params['be0'])
    cnt_raw = jax.ops.segment_sum(jnp.ones((N_EDGES,), jnp.float32), dst, n)
    cnt = jnp.maximum(cnt_raw, 1.0)
    for p in params['convs']:
        e = ea @ p['We'] + p['be']
        m = jnp.concatenate([h[dst_s], h[src_s], e], axis=-1) @ p['Wpre'] + p['bpre']
        m_padded = jnp.concatenate(
            [m, jnp.zeros((_MPAD - N_EDGES, HID), jnp.float32)])
        mn_k, mx_k, S_k, Q_k = _seg_agg(m_padded, rs)
        mean = jax.ops.segment_sum(m, dst_s, n) / cnt[:, None]
        has = (cnt_raw > 0)[:, None]
        mn = jnp.where(has, mn_k[:n], 0.0)
        mx = jnp.where(has, mx_k[:n], 0.0)
        msq = jax.ops.segment_sum(m * m, dst_s, n) / cnt[:, None]
        std = jnp.sqrt(jax.nn.relu(msq - mean * mean) + 1e-5)
        agg = jnp.concatenate([mean, mn, mx, std], axis=-1)
        logd = jnp.log(cnt + 1.0)[:, None]
        out = jnp.concatenate([agg, agg * (logd / AVG_LOG), agg * (AVG_LOG / logd)], axis=-1)
        out = jnp.concatenate([h, out], axis=-1) @ p['Wpost'] + p['bpost']
        out = out @ p['Wlin'] + p['blin']
        h = jax.nn.relu(_bn(out, p['g'], p['bb']))
    g = jax.ops.segment_sum(h, batch, NUM_GRAPHS)
    g = jax.nn.relu(_bn(g @ params['Wm'] + params['bm'], params['gm'], params['bmb']))
    g = jax.nn.relu(_bn(g @ params['Wo1'] + params['bo1'], params['go1'], params['bo1b']))
    g = jax.nn.relu(_bn(g @ params['Wo2'] + params['bo2'], params['go2'], params['bo2b']))
    return g @ params['Wo3'] + params['bo3']


# speedup vs baseline: 1.3889x; 1.3889x over previous
import functools
import numpy as np

import jax
import jax.numpy as jnp
from jax import lax
from jax.experimental import pallas as pl
from jax.experimental.pallas import tpu as pltpu
from jax.experimental.pallas import tpu_sc as plsc

N_NODES = 50000
N_EDGES = 800000
HID = 64
NUM_GRAPHS = 256
_DEG_HIST = np.array([0, 0, 0, 0, 120, 360, 840, 1600, 2600, 3600, 4400, 4900,
                      5000, 4900, 4400, 3600, 2600, 1600, 840, 360, 120],
                     dtype=np.float64)
AVG_LOG = float((np.log(np.arange(len(_DEG_HIST)) + 1.0) * _DEG_HIST).sum()
                / _DEG_HIST.sum())

_NW = 32
_NV = 1600
_NPAD = _NW * _NV
_RSBUF = 1664
_BLK = 32
_STAGE = 64
_MPAD = N_EDGES + 2 * _BLK + 8


def _seg_agg_body(m_hbm, rs_hbm, mn_hbm, mx_hbm, s_hbm, q_hbm,
                  rows_v, rs_v, mnst, mxst, sst, qst):
    wid = lax.axis_index("s") * 2 + lax.axis_index("c")
    vbase = wid * _NV
    pltpu.sync_copy(rs_hbm.at[pl.ds(vbase, _RSBUF)], rs_v)

    inf = jnp.full((16,), jnp.inf, jnp.float32)
    zero = jnp.zeros((16,), jnp.float32)

    def node_body(v_local, _):
        rsvec = rs_v[pl.ds(v_local, 16)]
        e0 = rsvec[0]
        e1 = rsvec[1]
        eb0 = pl.multiple_of((e0 // 8) * 8, 8)
        nblk = jnp.maximum(e1 - eb0 + (_BLK - 1), 0) // _BLK

        def blk_body(k, acc):
            eb = pl.multiple_of(eb0 + k * _BLK, 8)
            jstart = jnp.maximum(e0 - eb, 0)
            jend = jnp.minimum(e1 - eb, _BLK)
            pltpu.sync_copy(m_hbm.at[pl.ds(eb, _BLK)], rows_v)

            def edge_body(j, acc):
                (mn0, mn1, mn2, mn3, mx0, mx1, mx2, mx3,
                 s0, s1, s2, s3, q0, q1, q2, q3) = acc
                r0 = rows_v[j, pl.ds(0, 16)]
                r1 = rows_v[j, pl.ds(16, 16)]
                r2 = rows_v[j, pl.ds(32, 16)]
                r3 = rows_v[j, pl.ds(48, 16)]
                return (jnp.minimum(mn0, r0), jnp.minimum(mn1, r1),
                        jnp.minimum(mn2, r2), jnp.minimum(mn3, r3),
                        jnp.maximum(mx0, r0), jnp.maximum(mx1, r1),
                        jnp.maximum(mx2, r2), jnp.maximum(mx3, r3),
                        s0 + r0, s1 + r1, s2 + r2, s3 + r3,
                        q0 + r0 * r0, q1 + r1 * r1,
                        q2 + r2 * r2, q3 + r3 * r3)

            return lax.fori_loop(jstart, jend, edge_body, acc)

        init = (inf, inf, inf, inf, -inf, -inf, -inf, -inf,
                zero, zero, zero, zero, zero, zero, zero, zero)
        acc = lax.fori_loop(0, nblk, blk_body, init)

        sr = lax.rem(v_local, _STAGE)
        for k in range(4):
            mnst[sr, pl.ds(16 * k, 16)] = acc[k]
            mxst[sr, pl.ds(16 * k, 16)] = acc[4 + k]
            sst[sr, pl.ds(16 * k, 16)] = acc[8 + k]
            qst[sr, pl.ds(16 * k, 16)] = acc[12 + k]

        @pl.when(sr == _STAGE - 1)
        def _flush():
            row0 = pl.multiple_of(vbase + v_local - (_STAGE - 1), 8)
            pltpu.sync_copy(mnst, mn_hbm.at[pl.ds(row0, _STAGE)])
            pltpu.sync_copy(mxst, mx_hbm.at[pl.ds(row0, _STAGE)])
            pltpu.sync_copy(sst, s_hbm.at[pl.ds(row0, _STAGE)])
            pltpu.sync_copy(qst, q_hbm.at[pl.ds(row0, _STAGE)])

        return _

    lax.fori_loop(0, _NV, node_body, 0)


def _seg_agg(m_sorted, rs_padded):
    out = jax.ShapeDtypeStruct((_NPAD, HID), jnp.float32)
    kern = pl.kernel(
        _seg_agg_body,
        out_type=(out, out, out, out),
        mesh=plsc.VectorSubcoreMesh(core_axis_name="c", subcore_axis_name="s"),
        scratch_types=[
            pltpu.VMEM((_BLK, HID), jnp.float32),
            pltpu.VMEM((_RSBUF,), jnp.int32),
            pltpu.VMEM((_STAGE, HID), jnp.float32),
            pltpu.VMEM((_STAGE, HID), jnp.float32),
            pltpu.VMEM((_STAGE, HID), jnp.float32),
            pltpu.VMEM((_STAGE, HID), jnp.float32),
        ],
    )
    return kern(m_sorted, rs_padded)


def _csr(dst):
    perm = jnp.argsort(dst, stable=True).astype(jnp.int32)
    dst_sorted = dst[perm]
    rs = jnp.searchsorted(dst_sorted,
                          jnp.arange(_NPAD + _RSBUF, dtype=jnp.int32),
                          side='left').astype(jnp.int32)
    return perm, dst_sorted, rs


def _bn(x, g, b):
    mu = x.mean(0)
    v = x.var(0)
    return (x - mu) / jnp.sqrt(v + 1e-5) * g + b


def kernel(x, edge_index, edge_attr, batch, params):
    src, dst = edge_index[0], edge_index[1]
    n = N_NODES
    perm, dst_s, rs = _csr(dst)
    src_s = src[perm]
    h = jax.nn.leaky_relu(x @ params['Wn'] + params['bn'])
    ea = jax.nn.leaky_relu(edge_attr[perm] @ params['We0'] + params['be0'])
    cnt_raw = (rs[1:n + 1] - rs[:n]).astype(jnp.float32)
    cnt = jnp.maximum(cnt_raw, 1.0)
    for p in params['convs']:
        e = ea @ p['We'] + p['be']
        m = jnp.concatenate([h[dst_s], h[src_s], e], axis=-1) @ p['Wpre'] + p['bpre']
        m_padded = jnp.concatenate(
            [m, jnp.zeros((_MPAD - N_EDGES, HID), jnp.float32)])
        mn_k, mx_k, S_k, Q_k = _seg_agg(m_padded, rs)
        mean = S_k[:n] / cnt[:, None]
        has = (cnt_raw > 0)[:, None]
        mn = jnp.where(has, mn_k[:n], 0.0)
        mx = jnp.where(has, mx_k[:n], 0.0)
        msq = Q_k[:n] / cnt[:, None]
        std = jnp.sqrt(jax.nn.relu(msq - mean * mean) + 1e-5)
        agg = jnp.concatenate([mean, mn, mx, std], axis=-1)
        logd = jnp.log(cnt + 1.0)[:, None]
        out = jnp.concatenate([agg, agg * (logd / AVG_LOG), agg * (AVG_LOG / logd)], axis=-1)
        out = jnp.concatenate([h, out], axis=-1) @ p['Wpost'] + p['bpost']
        out = out @ p['Wlin'] + p['blin']
        h = jax.nn.relu(_bn(out, p['g'], p['bb']))
    g = jax.ops.segment_sum(h, batch, NUM_GRAPHS)
    g = jax.nn.relu(_bn(g @ params['Wm'] + params['bm'], params['gm'], params['bmb']))
    g = jax.nn.relu(_bn(g @ params['Wo1'] + params['bo1'], params['go1'], params['bo1b']))
    g = jax.nn.relu(_bn(g @ params['Wo2'] + params['bo2'], params['go2'], params['bo2b']))
    return g @ params['Wo3'] + params['bo3']
